# Initial kernel scaffold; baseline (speedup 1.0000x reference)
import jax
import jax.numpy as jnp
from jax.experimental import pallas as pl


def kernel(x, W1, W2, W3, fc_w, fc_b):
    raise NotImplementedError("write your pallas kernel here")


# speedup vs baseline: 3.3809x; 3.3809x over previous
import jax
import jax.numpy as jnp
from jax import lax
from jax.experimental import pallas as pl
from jax.experimental.pallas import tpu as pltpu

_R = 256
_KNN = 20


def _body(x_ref, xt_ref, xb_ref, at_ref, ct_ref, w2t_ref, w3t_ref,
          fct_ref, fcb_ref, out_ref, acc_ref):
    b = pl.program_id(0)
    r = pl.program_id(1)
    nb = pl.num_programs(1)
    n = x_ref.shape[2]

    xall = x_ref[0]
    xtall = xt_ref[0]
    xb = xb_ref[0]

    gram = jnp.dot(xb, xall, preferred_element_type=jnp.float32)
    xxc = jnp.sum(xall * xall, axis=0)
    xxr = jnp.sum(xb * xb, axis=1)
    dist = 2.0 * gram - xxr[:, None] - xxc[None, :]

    vb = jnp.dot(xb, ct_ref[...], preferred_element_type=jnp.float32)

    lane_iota = lax.broadcasted_iota(jnp.int32, (_R, n), 1)

    @pl.when(r == 0)
    def _():
        acc_ref[...] = jnp.full_like(acc_ref[...], -jnp.inf)

    def step(t, carry):
        dist, acc = carry
        m = jnp.max(dist, axis=1)
        eq = dist == m[:, None]
        am = jnp.min(jnp.where(eq, lane_iota, n), axis=1)
        onehot = lane_iota == am[:, None]
        ohf = onehot.astype(jnp.float32)
        xj = jnp.dot(ohf, xtall, preferred_element_type=jnp.float32)
        h1 = jnp.dot(xj, at_ref[...], preferred_element_type=jnp.float32) + vb
        h2 = jnp.maximum(
            jnp.dot(h1, w2t_ref[...], preferred_element_type=jnp.float32), 0.0)
        h3 = jnp.maximum(
            jnp.dot(h2, w3t_ref[...], preferred_element_type=jnp.float32), 0.0)
        acc = jnp.maximum(acc, jnp.max(h3, axis=0)[None, :])
        dist = jnp.where(onehot, -jnp.inf, dist)
        return dist, acc

    dist, acc = lax.fori_loop(0, _KNN, step, (dist, acc_ref[...]))
    acc_ref[...] = acc

    @pl.when(r == nb - 1)
    def _():
        logits = jnp.dot(acc_ref[...], fct_ref[...],
                         preferred_element_type=jnp.float32) + fcb_ref[...]
        out_ref[pl.ds(b, 1), :] = logits


def kernel(x, W1, W2, W3, fc_w, fc_b):
    B, D, N = x.shape
    ncls = fc_w.shape[0]
    xt = jnp.transpose(x, (0, 2, 1))
    A_T = jnp.transpose(W1[:, :D])
    C_T = jnp.transpose(W1[:, D:] - W1[:, :D])
    W2T = W2.T
    W3T = W3.T
    fcT = fc_w.T
    fcb = fc_b[None, :]

    grid = (B, N // _R)
    return pl.pallas_call(
        _body,
        grid=grid,
        in_specs=[
            pl.BlockSpec((1, D, N), lambda b, r: (b, 0, 0)),
            pl.BlockSpec((1, N, D), lambda b, r: (b, 0, 0)),
            pl.BlockSpec((1, _R, D), lambda b, r: (b, r, 0)),
            pl.BlockSpec((D, 64), lambda b, r: (0, 0)),
            pl.BlockSpec((D, 64), lambda b, r: (0, 0)),
            pl.BlockSpec((64, 128), lambda b, r: (0, 0)),
            pl.BlockSpec((128, 256), lambda b, r: (0, 0)),
            pl.BlockSpec((256, ncls), lambda b, r: (0, 0)),
            pl.BlockSpec((1, ncls), lambda b, r: (0, 0)),
        ],
        out_specs=pl.BlockSpec((B, ncls), lambda b, r: (0, 0)),
        out_shape=jax.ShapeDtypeStruct((B, ncls), jnp.float32),
        scratch_shapes=[pltpu.VMEM((1, 256), jnp.float32)],
    )(x, xt, xt, A_T, C_T, W2T, W3T, fcT, fcb)


# speedup vs baseline: 3.6008x; 1.0650x over previous
import jax
import jax.numpy as jnp
from jax import lax
from jax.experimental import pallas as pl
from jax.experimental.pallas import tpu as pltpu

_R = 256
_KNN = 20


def _body(x_ref, xt_ref, xb_ref, at_ref, ct_ref, w2t_ref, w3t_ref,
          fct_ref, fcb_ref, out_ref, acc_ref, xjs_ref):
    b = pl.program_id(0)
    r = pl.program_id(1)
    nb = pl.num_programs(1)
    n = x_ref.shape[2]

    xall = x_ref[0]
    xtall = xt_ref[0]
    xb = xb_ref[0]

    gram = jnp.dot(xb, xall, preferred_element_type=jnp.float32)
    xxc = jnp.sum(xall * xall, axis=0)
    xxr = jnp.sum(xb * xb, axis=1)
    dist = 2.0 * gram - xxr[:, None] - xxc[None, :]

    vb = jnp.dot(xb, ct_ref[...], preferred_element_type=jnp.float32)

    lane_iota = lax.broadcasted_iota(jnp.int32, (_R, n), 1)

    @pl.when(r == 0)
    def _():
        acc_ref[...] = jnp.full_like(acc_ref[...], -jnp.inf)

    def step(t, dist):
        am = jnp.argmax(dist, axis=1)
        ohf = (lane_iota == am[:, None]).astype(jnp.float32)
        xj = jnp.dot(ohf, xtall, preferred_element_type=jnp.float32)
        xjs_ref[pl.ds(t, 1), :, :] = xj[None]
        dist = dist - ohf * jnp.float32(3e38)
        return dist

    dist = lax.fori_loop(0, _KNN, step, dist)

    xe = xjs_ref[...].reshape(_KNN * _R, 3)
    h1 = jnp.dot(xe, at_ref[...], preferred_element_type=jnp.float32)
    h1 = h1 + jnp.broadcast_to(vb[None], (_KNN, _R, 64)).reshape(_KNN * _R, 64)
    h2 = jnp.maximum(
        jnp.dot(h1, w2t_ref[...], preferred_element_type=jnp.float32), 0.0)
    h3 = jnp.maximum(
        jnp.dot(h2, w3t_ref[...], preferred_element_type=jnp.float32), 0.0)
    bm = jnp.max(h3, axis=0)[None, :]
    acc_ref[...] = jnp.maximum(acc_ref[...], bm)

    @pl.when(r == nb - 1)
    def _():
        logits = jnp.dot(acc_ref[...], fct_ref[...],
                         preferred_element_type=jnp.float32) + fcb_ref[...]
        out_ref[pl.ds(b, 1), :] = logits


def kernel(x, W1, W2, W3, fc_w, fc_b):
    B, D, N = x.shape
    ncls = fc_w.shape[0]
    xt = jnp.transpose(x, (0, 2, 1))
    A_T = jnp.transpose(W1[:, :D])
    C_T = jnp.transpose(W1[:, D:] - W1[:, :D])
    W2T = W2.T
    W3T = W3.T
    fcT = fc_w.T
    fcb = fc_b[None, :]

    grid = (B, N // _R)
    return pl.pallas_call(
        _body,
        grid=grid,
        in_specs=[
            pl.BlockSpec((1, D, N), lambda b, r: (b, 0, 0)),
            pl.BlockSpec((1, N, D), lambda b, r: (b, 0, 0)),
            pl.BlockSpec((1, _R, D), lambda b, r: (b, r, 0)),
            pl.BlockSpec((D, 64), lambda b, r: (0, 0)),
            pl.BlockSpec((D, 64), lambda b, r: (0, 0)),
            pl.BlockSpec((64, 128), lambda b, r: (0, 0)),
            pl.BlockSpec((128, 256), lambda b, r: (0, 0)),
            pl.BlockSpec((256, ncls), lambda b, r: (0, 0)),
            pl.BlockSpec((1, ncls), lambda b, r: (0, 0)),
        ],
        out_specs=pl.BlockSpec((B, ncls), lambda b, r: (0, 0)),
        out_shape=jax.ShapeDtypeStruct((B, ncls), jnp.float32),
        scratch_shapes=[pltpu.VMEM((1, 256), jnp.float32),
                        pltpu.VMEM((_KNN, _R, 3), jnp.float32)],
    )(x, xt, xt, A_T, C_T, W2T, W3T, fcT, fcb)


# speedup vs baseline: 3.9693x; 1.1023x over previous
import jax
import jax.numpy as jnp
from jax import lax
from jax.experimental import pallas as pl
from jax.experimental.pallas import tpu as pltpu

_R = 256
_KNN = 20


def _body(x_ref, xg_ref, xb_ref, at_ref, ct_ref, w2t_ref, w3t_ref,
          fct_ref, fcb_ref, out_ref, acc_ref, xjs_ref):
    b = pl.program_id(0)
    r = pl.program_id(1)
    nb = pl.num_programs(1)
    n = x_ref.shape[2]

    xall = x_ref[0]
    xb = xb_ref[0]
    nch = n // 128

    gram = jnp.dot(xb, xall, preferred_element_type=jnp.float32)
    xxc = jnp.sum(xall * xall, axis=0)
    xxr = jnp.sum(xb * xb, axis=1)
    dist = 2.0 * gram - xxr[:, None] - xxc[None, :]

    vb = jnp.dot(xb, ct_ref[...], preferred_element_type=jnp.float32)

    lane_iota = lax.broadcasted_iota(jnp.int32, (_R, n), 1)

    @pl.when(r == 0)
    def _():
        acc_ref[...] = jnp.full_like(acc_ref[...], -jnp.inf)

    chunk_iota = lax.broadcasted_iota(jnp.int32, (_R, nch), 1)
    l_iota = lax.broadcasted_iota(jnp.int32, (_R, 128), 1)

    def step(t, dist):
        am = jnp.argmax(dist, axis=1)
        dist = jnp.where(lane_iota == am[:, None], jnp.float32(-3e38), dist)
        c = jnp.right_shift(am, 7)
        lane = jnp.bitwise_and(am, 127)
        ohc = (chunk_iota == c[:, None]).astype(jnp.float32)
        ohl = (l_iota == lane[:, None]).astype(jnp.float32)
        cols = []
        for d in range(3):
            rc = jnp.dot(ohc, xg_ref[0, d],
                         preferred_element_type=jnp.float32)
            cols.append(jnp.sum(rc * ohl, axis=1)[:, None])
        xjs_ref[pl.ds(t, 1), :, :] = jnp.concatenate(cols, axis=1)[None]
        return dist

    dist = lax.fori_loop(0, _KNN, step, dist)

    xe = xjs_ref[...].reshape(_KNN * _R, 3)
    h1 = jnp.dot(xe, at_ref[...], preferred_element_type=jnp.float32)
    h1 = h1 + jnp.broadcast_to(vb[None], (_KNN, _R, 64)).reshape(_KNN * _R, 64)
    h2 = jnp.maximum(
        jnp.dot(h1, w2t_ref[...], preferred_element_type=jnp.float32), 0.0)
    h3 = jnp.maximum(
        jnp.dot(h2, w3t_ref[...], preferred_element_type=jnp.float32), 0.0)
    bm = jnp.max(h3, axis=0)[None, :]
    acc_ref[...] = jnp.maximum(acc_ref[...], bm)

    @pl.when(r == nb - 1)
    def _():
        logits = jnp.dot(acc_ref[...], fct_ref[...],
                         preferred_element_type=jnp.float32) + fcb_ref[...]
        out_ref[pl.ds(b, 1), :] = logits


def kernel(x, W1, W2, W3, fc_w, fc_b):
    B, D, N = x.shape
    ncls = fc_w.shape[0]
    xt = jnp.transpose(x, (0, 2, 1))
    A_T = jnp.transpose(W1[:, :D])
    C_T = jnp.transpose(W1[:, D:] - W1[:, :D])
    W2T = W2.T
    W3T = W3.T
    fcT = fc_w.T
    fcb = fc_b[None, :]

    grid = (B, N // _R)
    return pl.pallas_call(
        _body,
        grid=grid,
        in_specs=[
            pl.BlockSpec((1, D, N), lambda b, r: (b, 0, 0)),
            pl.BlockSpec((1, D, N // 128, 128), lambda b, r: (b, 0, 0, 0)),
            pl.BlockSpec((1, _R, D), lambda b, r: (b, r, 0)),
            pl.BlockSpec((D, 64), lambda b, r: (0, 0)),
            pl.BlockSpec((D, 64), lambda b, r: (0, 0)),
            pl.BlockSpec((64, 128), lambda b, r: (0, 0)),
            pl.BlockSpec((128, 256), lambda b, r: (0, 0)),
            pl.BlockSpec((256, ncls), lambda b, r: (0, 0)),
            pl.BlockSpec((1, ncls), lambda b, r: (0, 0)),
        ],
        out_specs=pl.BlockSpec((B, ncls), lambda b, r: (0, 0)),
        out_shape=jax.ShapeDtypeStruct((B, ncls), jnp.float32),
        scratch_shapes=[pltpu.VMEM((1, 256), jnp.float32),
                        pltpu.VMEM((_KNN, _R, 3), jnp.float32)],
    )(x, x.reshape(B, D, N // 128, 128), xt, A_T, C_T, W2T, W3T, fcT, fcb)


# speedup vs baseline: 5.3244x; 1.3414x over previous
import jax
import jax.numpy as jnp
from jax import lax
from jax.experimental import pallas as pl
from jax.experimental.pallas import tpu as pltpu

_R = 512
_KNN = 20


def _body(x_ref, xg_ref, xb_ref, at_ref, ct_ref, w2t_ref, w3t_ref,
          fct_ref, fcb_ref, out_ref, acc_ref, xjs_ref):
    b = pl.program_id(0)
    r = pl.program_id(1)
    nb = pl.num_programs(1)
    n = x_ref.shape[2]

    xall = x_ref[0]
    xb = xb_ref[0]
    nch = n // 128

    gram = jnp.dot(xb, xall, preferred_element_type=jnp.float32)
    xxc = jnp.sum(xall * xall, axis=0)
    xxr = jnp.sum(xb * xb, axis=1)
    dist = 2.0 * gram - xxr[:, None] - xxc[None, :]

    vb = jnp.dot(xb, ct_ref[...], preferred_element_type=jnp.float32)

    lane_iota = lax.broadcasted_iota(jnp.int32, (_R, n), 1)

    @pl.when(r == 0)
    def _():
        acc_ref[...] = jnp.full_like(acc_ref[...], -jnp.inf)

    chunk_iota = lax.broadcasted_iota(jnp.int32, (_R, nch), 1)
    l_iota = lax.broadcasted_iota(jnp.int32, (_R, 128), 1)

    def one_extract(t, dist):
        am = jnp.argmax(dist, axis=1)
        dist = jnp.where(lane_iota == am[:, None], jnp.float32(-3e38), dist)
        c = jnp.right_shift(am, 7)
        lane = jnp.bitwise_and(am, 127)
        ohc = (chunk_iota == c[:, None]).astype(jnp.float32)
        ohl = (l_iota == lane[:, None]).astype(jnp.float32)
        cols = []
        for d in range(3):
            rc = jnp.dot(ohc, xg_ref[0, d],
                         preferred_element_type=jnp.float32)
            cols.append(jnp.sum(rc * ohl, axis=1)[:, None])
        xjs_ref[pl.ds(t, 1), :, :] = jnp.concatenate(cols, axis=1)[None]
        return dist

    def step(t2, dist):
        dist = one_extract(2 * t2, dist)
        dist = one_extract(2 * t2 + 1, dist)
        return dist

    dist = lax.fori_loop(0, _KNN // 2, step, dist)

    xe = xjs_ref[...].reshape(_KNN * _R, 3)
    h1 = jnp.dot(xe, at_ref[...], preferred_element_type=jnp.float32)
    h1 = h1 + jnp.broadcast_to(vb[None], (_KNN, _R, 64)).reshape(_KNN * _R, 64)
    h2 = jnp.maximum(
        jnp.dot(h1, w2t_ref[...], preferred_element_type=jnp.float32), 0.0)
    h3 = jnp.maximum(
        jnp.dot(h2, w3t_ref[...], preferred_element_type=jnp.float32), 0.0)
    bm = jnp.max(h3, axis=0)[None, :]
    acc_ref[...] = jnp.maximum(acc_ref[...], bm)

    @pl.when(r == nb - 1)
    def _():
        logits = jnp.dot(acc_ref[...], fct_ref[...],
                         preferred_element_type=jnp.float32) + fcb_ref[...]
        out_ref[pl.ds(b, 1), :] = logits


def kernel(x, W1, W2, W3, fc_w, fc_b):
    B, D, N = x.shape
    ncls = fc_w.shape[0]
    xt = jnp.transpose(x, (0, 2, 1))
    A_T = jnp.transpose(W1[:, :D])
    C_T = jnp.transpose(W1[:, D:] - W1[:, :D])
    W2T = W2.T
    W3T = W3.T
    fcT = fc_w.T
    fcb = fc_b[None, :]

    grid = (B, N // _R)
    return pl.pallas_call(
        _body,
        grid=grid,
        in_specs=[
            pl.BlockSpec((1, D, N), lambda b, r: (b, 0, 0)),
            pl.BlockSpec((1, D, N // 128, 128), lambda b, r: (b, 0, 0, 0)),
            pl.BlockSpec((1, _R, D), lambda b, r: (b, r, 0)),
            pl.BlockSpec((D, 64), lambda b, r: (0, 0)),
            pl.BlockSpec((D, 64), lambda b, r: (0, 0)),
            pl.BlockSpec((64, 128), lambda b, r: (0, 0)),
            pl.BlockSpec((128, 256), lambda b, r: (0, 0)),
            pl.BlockSpec((256, ncls), lambda b, r: (0, 0)),
            pl.BlockSpec((1, ncls), lambda b, r: (0, 0)),
        ],
        out_specs=pl.BlockSpec((B, ncls), lambda b, r: (0, 0)),
        out_shape=jax.ShapeDtypeStruct((B, ncls), jnp.float32),
        scratch_shapes=[pltpu.VMEM((1, 256), jnp.float32),
                        pltpu.VMEM((_KNN, _R, 3), jnp.float32)],
    )(x, x.reshape(B, D, N // 128, 128), xt, A_T, C_T, W2T, W3T, fcT, fcb)


# speedup vs baseline: 5.8338x; 1.0957x over previous
import jax
import jax.numpy as jnp
from jax import lax
from jax.experimental import pallas as pl
from jax.experimental.pallas import tpu as pltpu

_R = 1024
_KNN = 20


def _body(x_ref, xg_ref, xb_ref, at_ref, ct_ref, w2t_ref, w3t_ref,
          fct_ref, fcb_ref, out_ref, acc_ref, xjs_ref):
    b = pl.program_id(0)
    r = pl.program_id(1)
    nb = pl.num_programs(1)
    n = x_ref.shape[2]

    xall = x_ref[0]
    xb = xb_ref[0]
    nch = n // 128

    gram = jnp.dot(xb, xall, preferred_element_type=jnp.float32)
    xxc = jnp.sum(xall * xall, axis=0)
    xxr = jnp.sum(xb * xb, axis=1)
    dist = 2.0 * gram - xxr[:, None] - xxc[None, :]

    vb = jnp.dot(xb, ct_ref[...], preferred_element_type=jnp.float32)

    lane_iota = lax.broadcasted_iota(jnp.int32, (_R, n), 1)

    @pl.when(r == 0)
    def _():
        acc_ref[...] = jnp.full_like(acc_ref[...], -jnp.inf)

    chunk_iota = lax.broadcasted_iota(jnp.int32, (_R, nch), 1)
    l_iota = lax.broadcasted_iota(jnp.int32, (_R, 128), 1)

    def one_extract(t, dist):
        am = jnp.argmax(dist, axis=1)
        dist = jnp.where(lane_iota == am[:, None], jnp.float32(-3e38), dist)
        c = jnp.right_shift(am, 7)
        lane = jnp.bitwise_and(am, 127)
        ohc = (chunk_iota == c[:, None]).astype(jnp.float32)
        ohl = (l_iota == lane[:, None]).astype(jnp.float32)
        cols = []
        for d in range(3):
            rc = jnp.dot(ohc, xg_ref[0, d],
                         preferred_element_type=jnp.float32)
            cols.append(jnp.sum(rc * ohl, axis=1)[:, None])
        xjs_ref[pl.ds(t, 1), :, :] = jnp.concatenate(cols, axis=1)[None]
        return dist

    def step(t2, dist):
        dist = one_extract(2 * t2, dist)
        dist = one_extract(2 * t2 + 1, dist)
        return dist

    dist = lax.fori_loop(0, _KNN // 2, step, dist)

    xe = xjs_ref[...].reshape(_KNN * _R, 3)
    h1 = jnp.dot(xe, at_ref[...], preferred_element_type=jnp.float32)
    h1 = h1 + jnp.broadcast_to(vb[None], (_KNN, _R, 64)).reshape(_KNN * _R, 64)
    h2 = jnp.maximum(
        jnp.dot(h1, w2t_ref[...], preferred_element_type=jnp.float32), 0.0)
    h3 = jnp.maximum(
        jnp.dot(h2, w3t_ref[...], preferred_element_type=jnp.float32), 0.0)
    bm = jnp.max(h3, axis=0)[None, :]
    acc_ref[...] = jnp.maximum(acc_ref[...], bm)

    @pl.when(r == nb - 1)
    def _():
        logits = jnp.dot(acc_ref[...], fct_ref[...],
                         preferred_element_type=jnp.float32) + fcb_ref[...]
        out_ref[pl.ds(b, 1), :] = logits


def kernel(x, W1, W2, W3, fc_w, fc_b):
    B, D, N = x.shape
    ncls = fc_w.shape[0]
    xt = jnp.transpose(x, (0, 2, 1))
    A_T = jnp.transpose(W1[:, :D])
    C_T = jnp.transpose(W1[:, D:] - W1[:, :D])
    W2T = W2.T
    W3T = W3.T
    fcT = fc_w.T
    fcb = fc_b[None, :]

    grid = (B, N // _R)
    return pl.pallas_call(
        _body,
        grid=grid,
        in_specs=[
            pl.BlockSpec((1, D, N), lambda b, r: (b, 0, 0)),
            pl.BlockSpec((1, D, N // 128, 128), lambda b, r: (b, 0, 0, 0)),
            pl.BlockSpec((1, _R, D), lambda b, r: (b, r, 0)),
            pl.BlockSpec((D, 64), lambda b, r: (0, 0)),
            pl.BlockSpec((D, 64), lambda b, r: (0, 0)),
            pl.BlockSpec((64, 128), lambda b, r: (0, 0)),
            pl.BlockSpec((128, 256), lambda b, r: (0, 0)),
            pl.BlockSpec((256, ncls), lambda b, r: (0, 0)),
            pl.BlockSpec((1, ncls), lambda b, r: (0, 0)),
        ],
        out_specs=pl.BlockSpec((B, ncls), lambda b, r: (0, 0)),
        out_shape=jax.ShapeDtypeStruct((B, ncls), jnp.float32),
        scratch_shapes=[pltpu.VMEM((1, 256), jnp.float32),
                        pltpu.VMEM((_KNN, _R, 3), jnp.float32)],
    )(x, x.reshape(B, D, N // 128, 128), xt, A_T, C_T, W2T, W3T, fcT, fcb)


# speedup vs baseline: 6.2226x; 1.0666x over previous
import jax
import jax.numpy as jnp
from jax import lax
from jax.experimental import pallas as pl
from jax.experimental.pallas import tpu as pltpu

_R = 2048
_KNN = 20


def _body(x_ref, xg_ref, xb_ref, at_ref, ct_ref, w2t_ref, w3t_ref,
          fct_ref, fcb_ref, out_ref, acc_ref, xjs_ref):
    b = pl.program_id(0)
    r = pl.program_id(1)
    nb = pl.num_programs(1)
    n = x_ref.shape[2]

    xall = x_ref[0]
    xb = xb_ref[0]
    nch = n // 128

    gram = jnp.dot(xb, xall, preferred_element_type=jnp.float32)
    xxc = jnp.sum(xall * xall, axis=0)
    xxr = jnp.sum(xb * xb, axis=1)
    dist = 2.0 * gram - xxr[:, None] - xxc[None, :]

    vb = jnp.dot(xb, ct_ref[...], preferred_element_type=jnp.float32)

    lane_iota = lax.broadcasted_iota(jnp.int32, (_R, n), 1)

    @pl.when(r == 0)
    def _():
        acc_ref[...] = jnp.full_like(acc_ref[...], -jnp.inf)

    chunk_iota = lax.broadcasted_iota(jnp.int32, (_R, nch), 1)
    l_iota = lax.broadcasted_iota(jnp.int32, (_R, 128), 1)

    def one_extract(t, dist):
        am = jnp.argmax(dist, axis=1)
        dist = jnp.where(lane_iota == am[:, None], jnp.float32(-3e38), dist)
        c = jnp.right_shift(am, 7)
        lane = jnp.bitwise_and(am, 127)
        ohc = (chunk_iota == c[:, None]).astype(jnp.float32)
        ohl = (l_iota == lane[:, None]).astype(jnp.float32)
        cols = []
        for d in range(3):
            rc = jnp.dot(ohc, xg_ref[0, d],
                         preferred_element_type=jnp.float32)
            cols.append(jnp.sum(rc * ohl, axis=1)[:, None])
        xjs_ref[pl.ds(t, 1), :, :] = jnp.concatenate(cols, axis=1)[None]
        return dist

    def step(t2, dist):
        dist = one_extract(2 * t2, dist)
        dist = one_extract(2 * t2 + 1, dist)
        return dist

    dist = lax.fori_loop(0, _KNN // 2, step, dist)

    xe = xjs_ref[...].reshape(_KNN * _R, 3)
    h1 = jnp.dot(xe, at_ref[...], preferred_element_type=jnp.float32)
    h1 = h1 + jnp.broadcast_to(vb[None], (_KNN, _R, 64)).reshape(_KNN * _R, 64)
    h2 = jnp.maximum(
        jnp.dot(h1, w2t_ref[...], preferred_element_type=jnp.float32), 0.0)
    h3 = jnp.maximum(
        jnp.dot(h2, w3t_ref[...], preferred_element_type=jnp.float32), 0.0)
    bm = jnp.max(h3, axis=0)[None, :]
    acc_ref[...] = jnp.maximum(acc_ref[...], bm)

    @pl.when(r == nb - 1)
    def _():
        logits = jnp.dot(acc_ref[...], fct_ref[...],
                         preferred_element_type=jnp.float32) + fcb_ref[...]
        out_ref[pl.ds(b, 1), :] = logits


def kernel(x, W1, W2, W3, fc_w, fc_b):
    B, D, N = x.shape
    ncls = fc_w.shape[0]
    xt = jnp.transpose(x, (0, 2, 1))
    A_T = jnp.transpose(W1[:, :D])
    C_T = jnp.transpose(W1[:, D:] - W1[:, :D])
    W2T = W2.T
    W3T = W3.T
    fcT = fc_w.T
    fcb = fc_b[None, :]

    grid = (B, N // _R)
    return pl.pallas_call(
        _body,
        grid=grid,
        in_specs=[
            pl.BlockSpec((1, D, N), lambda b, r: (b, 0, 0)),
            pl.BlockSpec((1, D, N // 128, 128), lambda b, r: (b, 0, 0, 0)),
            pl.BlockSpec((1, _R, D), lambda b, r: (b, r, 0)),
            pl.BlockSpec((D, 64), lambda b, r: (0, 0)),
            pl.BlockSpec((D, 64), lambda b, r: (0, 0)),
            pl.BlockSpec((64, 128), lambda b, r: (0, 0)),
            pl.BlockSpec((128, 256), lambda b, r: (0, 0)),
            pl.BlockSpec((256, ncls), lambda b, r: (0, 0)),
            pl.BlockSpec((1, ncls), lambda b, r: (0, 0)),
        ],
        out_specs=pl.BlockSpec((B, ncls), lambda b, r: (0, 0)),
        out_shape=jax.ShapeDtypeStruct((B, ncls), jnp.float32),
        scratch_shapes=[pltpu.VMEM((1, 256), jnp.float32),
                        pltpu.VMEM((_KNN, _R, 3), jnp.float32)],
    )(x, x.reshape(B, D, N // 128, 128), xt, A_T, C_T, W2T, W3T, fcT, fcb)


# speedup vs baseline: 7.0237x; 1.1287x over previous
import jax
import jax.numpy as jnp
from jax import lax
from jax.experimental import pallas as pl
from jax.experimental.pallas import tpu as pltpu

_R = 2048
_KNN = 20


def _body(x_ref, xg_ref, xb_ref, at_ref, ct_ref, w2t_ref, w3t_ref,
          fct_ref, fcb_ref, out_ref, acc_ref, xjs_ref):
    b = pl.program_id(0)
    r = pl.program_id(1)
    nb = pl.num_programs(1)
    n = x_ref.shape[2]

    xall = x_ref[0]
    xb = xb_ref[0]
    nch = n // 128

    gram = jnp.dot(xb, xall, preferred_element_type=jnp.float32)
    xxc = jnp.sum(xall * xall, axis=0)
    xxr = jnp.sum(xb * xb, axis=1)
    dist = 2.0 * gram - xxr[:, None] - xxc[None, :]

    vb = jnp.dot(xb, ct_ref[...], preferred_element_type=jnp.float32)

    lane_iota = lax.broadcasted_iota(jnp.int32, (_R, n), 1)

    @pl.when(r == 0)
    def _():
        acc_ref[...] = jnp.full_like(acc_ref[...], -jnp.inf)

    chunk_iota = lax.broadcasted_iota(jnp.int32, (_R, nch), 1)
    l_iota = lax.broadcasted_iota(jnp.int32, (_R, 128), 1)

    def one_extract(t, dist):
        am = jnp.argmax(dist, axis=1)
        dist = jnp.where(lane_iota == am[:, None], jnp.float32(-3e38), dist)
        c = jnp.right_shift(am, 7)
        lane = jnp.bitwise_and(am, 127)
        ohc = (chunk_iota == c[:, None]).astype(jnp.float32)
        ohl = (l_iota == lane[:, None]).astype(jnp.float32)
        cols = []
        for d in range(3):
            rc = jnp.dot(ohc, xg_ref[0, d],
                         preferred_element_type=jnp.float32)
            cols.append(jnp.sum(rc * ohl, axis=1)[:, None])
        xjs_ref[pl.ds(t, 1), :, :] = jnp.concatenate(cols, axis=1)[None]
        return dist

    for t in range(_KNN):
        dist = one_extract(t, dist)

    xe = xjs_ref[...].reshape(_KNN * _R, 3)
    h1 = jnp.dot(xe, at_ref[...], preferred_element_type=jnp.float32)
    h1 = h1 + jnp.broadcast_to(vb[None], (_KNN, _R, 64)).reshape(_KNN * _R, 64)
    h2 = jnp.maximum(
        jnp.dot(h1, w2t_ref[...], preferred_element_type=jnp.float32), 0.0)
    h3 = jnp.maximum(
        jnp.dot(h2, w3t_ref[...], preferred_element_type=jnp.float32), 0.0)
    bm = jnp.max(h3, axis=0)[None, :]
    acc_ref[...] = jnp.maximum(acc_ref[...], bm)

    @pl.when(r == nb - 1)
    def _():
        logits = jnp.dot(acc_ref[...], fct_ref[...],
                         preferred_element_type=jnp.float32) + fcb_ref[...]
        out_ref[pl.ds(b, 1), :] = logits


def kernel(x, W1, W2, W3, fc_w, fc_b):
    B, D, N = x.shape
    ncls = fc_w.shape[0]
    xt = jnp.transpose(x, (0, 2, 1))
    A_T = jnp.transpose(W1[:, :D])
    C_T = jnp.transpose(W1[:, D:] - W1[:, :D])
    W2T = W2.T
    W3T = W3.T
    fcT = fc_w.T
    fcb = fc_b[None, :]

    grid = (B, N // _R)
    return pl.pallas_call(
        _body,
        grid=grid,
        in_specs=[
            pl.BlockSpec((1, D, N), lambda b, r: (b, 0, 0)),
            pl.BlockSpec((1, D, N // 128, 128), lambda b, r: (b, 0, 0, 0)),
            pl.BlockSpec((1, _R, D), lambda b, r: (b, r, 0)),
            pl.BlockSpec((D, 64), lambda b, r: (0, 0)),
            pl.BlockSpec((D, 64), lambda b, r: (0, 0)),
            pl.BlockSpec((64, 128), lambda b, r: (0, 0)),
            pl.BlockSpec((128, 256), lambda b, r: (0, 0)),
            pl.BlockSpec((256, ncls), lambda b, r: (0, 0)),
            pl.BlockSpec((1, ncls), lambda b, r: (0, 0)),
        ],
        out_specs=pl.BlockSpec((B, ncls), lambda b, r: (0, 0)),
        out_shape=jax.ShapeDtypeStruct((B, ncls), jnp.float32),
        scratch_shapes=[pltpu.VMEM((1, 256), jnp.float32),
                        pltpu.VMEM((_KNN, _R, 3), jnp.float32)],
    )(x, x.reshape(B, D, N // 128, 128), xt, A_T, C_T, W2T, W3T, fcT, fcb)
